# baseline (speedup 1.0000x reference)
import functools

import jax
import jax.numpy as jnp
from jax import lax
from jax.experimental import pallas as pl
from jax.experimental.pallas import tpu as pltpu
from jax.experimental.pallas import tpu_sc as plsc

B = 16
S = 2048
H = 32
KVH = 8
DH = 128
SLOTS = 32768
SCALE = 0.08838834764831845
GROUP = H // KVH

C_SC = 32
NW = 32
C_TC = 256
NCHUNK = S // C_TC
HALF = S // 2
NCH = NCHUNK // 2
W = KVH * C_TC

NEG = -1e30


def _sc_gather(k_cache, v_cache, active_slots, lens, half):
    mesh = plsc.VectorSubcoreMesh(
        core_axis_name="c", subcore_axis_name="s", num_cores=2, num_subcores=16
    )
    out_sd = jax.ShapeDtypeStruct((B, HALF, KVH, DH), jnp.float32)

    @functools.partial(
        pl.kernel,
        out_type=[out_sd, out_sd],
        mesh=mesh,
        scratch_types=[
            pltpu.VMEM((B,), jnp.int32),
            pltpu.VMEM((C_SC,), jnp.int32),
            pltpu.VMEM((C_SC, KVH, DH), jnp.float32),
            pltpu.VMEM((C_SC, KVH, DH), jnp.float32),
            pltpu.SemaphoreType.DMA,
            pltpu.SemaphoreType.DMA,
            pltpu.SemaphoreType.DMA,
            pltpu.SemaphoreType.DMA,
        ],
        name=f"sc_gather_h{half}",
    )
    def sc_kernel(kc, vc, slots, lens_h, gk, gv,
                  lens_v, idx_v, rows0, rows1, sem_g0, sem_g1, sem_s0, sem_s1):
        wid = lax.axis_index("s") * 2 + lax.axis_index("c")
        pltpu.sync_copy(lens_h, lens_v)
        lens_vec = lens_v[...]

        items = [(b, cache) for b in range(B) for cache in (0, 1)]

        def active(i):
            bb = items[i][0]
            return (half * NW + wid) * C_SC < lens_vec[bb]

        rows = (rows0, rows1)
        sem_g = (sem_g0, sem_g1)
        sem_s = (sem_s0, sem_s1)

        def drain(par, dst):
            pltpu.make_async_copy(
                rows[par], dst.at[0, pl.ds(0, C_SC)], sem_s[par]).wait()

        for i, (b, cache) in enumerate(items):
            if i >= 2:
                @pl.when(active(i - 2))
                def _(par=cache, dst=(gk, gv)[cache]):
                    drain(par, dst)

            @pl.when(active(i))
            def _(b=b, cache=cache):
                jsrc = (half * NW + wid) * C_SC
                jdst = wid * C_SC
                if cache == 0:
                    pltpu.sync_copy(slots.at[b, pl.ds(jsrc, C_SC)], idx_v)
                src = (kc, vc)[cache]
                dst = (gk, gv)[cache]
                pltpu.async_copy(src.at[idx_v], rows[cache],
                                 sem_g[cache]).wait()
                pltpu.async_copy(rows[cache],
                                 dst.at[b, pl.ds(jdst, C_SC)],
                                 sem_s[cache])

        for i in (len(items) - 2, len(items) - 1):
            @pl.when(active(i))
            def _(par=items[i][1], dst=(gk, gv)[items[i][1]]):
                drain(par, dst)

    return sc_kernel(k_cache, v_cache, active_slots, lens)


def _make_partial_body(half):
    def body(lens_ref, nactm1_ref, q_ref, k_ref, v_ref,
             slotsr_ref, slotsc_ref, smc_ref, smr_ref,
             m_out, l_out, acc_out, cnt_out, m_scr, l_scr, acc_scr, cnt_scr):
        b = pl.program_id(0)
        c = pl.program_id(1)
        cg = c + half * NCH

        @pl.when(c == 0)
        def _():
            m_scr[...] = jnp.full((H, DH), NEG, jnp.float32)
            l_scr[...] = jnp.zeros((H, DH), jnp.float32)
            acc_scr[...] = jnp.zeros((H, DH), jnp.float32)
            cnt_scr[...] = jnp.zeros((B, DH), jnp.float32)

        @pl.when(cg <= nactm1_ref[b])
        def _():
            lb = lens_ref[b]
            q_all = q_ref[0]
            kflat = k_ref[0].reshape(W, DH)
            vflat = v_ref[0].reshape(W, DH)
            sm_c = smc_ref[...]
            sm_r = smr_ref[...]
            slots_row = slotsr_ref[0, 0]
            slots_col = slotsc_ref[0, 0]

            match16 = sm_c == slots_row
            validr = (cg * C_TC
                      + lax.broadcasted_iota(jnp.int32, (1, W), 1) // KVH) < lb
            cnt_add = jnp.sum(jnp.where(match16 & validr, 1.0 / KVH, 0.0),
                              axis=1, keepdims=True)
            cnt_scr[...] = cnt_scr[...] + jnp.broadcast_to(cnt_add, (B, DH))

            newr = jnp.max(match16.astype(jnp.float32), axis=0, keepdims=True)
            keep_row = jnp.logical_and(validr, newr < 0.5)
            hg = lax.broadcasted_iota(jnp.int32, (H, 1), 0) // GROUP
            jg = lax.broadcasted_iota(jnp.int32, (1, W), 1) % KVH
            smask = jnp.logical_and(hg == jg, keep_row)

            matchc = jnp.max((slots_col == sm_r).astype(jnp.float32),
                             axis=1, keepdims=True)
            validc = (cg * C_TC
                      + lax.broadcasted_iota(jnp.int32, (W, 1), 0) // KVH) < lb
            keep_c = jnp.logical_and(validc, matchc < 0.5)
            v_use = jnp.where(keep_c, vflat, 0.0)

            s = lax.dot_general(q_all, kflat, (((1,), (1,)), ((), ())),
                                preferred_element_type=jnp.float32) * SCALE
            s = jnp.where(smask, s, NEG)

            m_old = m_scr[:, 0:1]
            m_new = jnp.maximum(m_old, jnp.max(s, axis=1, keepdims=True))
            alpha = jnp.exp(m_old - m_new)
            p = jnp.where(smask, jnp.exp(s - m_new), 0.0)

            l_new = l_scr[:, 0:1] * alpha + jnp.sum(p, axis=1, keepdims=True)
            acc = acc_scr[...] * alpha + lax.dot_general(
                p, v_use, (((1,), (0,)), ((), ())),
                preferred_element_type=jnp.float32)

            m_scr[...] = jnp.broadcast_to(m_new, (H, DH))
            l_scr[...] = jnp.broadcast_to(l_new, (H, DH))
            acc_scr[...] = acc

        m_out[0] = m_scr[...]
        l_out[0] = l_scr[...]
        acc_out[0] = acc_scr[...]
        cnt_out[0] = cnt_scr[...]

    return body


def _tc_partial(q, gk, gv, slots4, slots4c, sm_c, sm_r, lens, nactm1, half):
    def q_map(b, c, lens_ref, nactm1_ref):
        return (b, 0, 0)

    def kv_map(b, c, lens_ref, nactm1_ref):
        lam = jnp.clip(nactm1_ref[b] - half * NCH, 0, NCH - 1)
        return (b, jnp.minimum(c, lam), 0, 0)

    def slots_map(b, c, lens_ref, nactm1_ref):
        lam = jnp.clip(nactm1_ref[b] - half * NCH, 0, NCH - 1)
        return (b, jnp.minimum(c, lam), 0, 0)

    def sm_map(b, c, lens_ref, nactm1_ref):
        return (0, 0)

    sd = jax.ShapeDtypeStruct((B, H, DH), jnp.float32)
    sdc = jax.ShapeDtypeStruct((B, B, DH), jnp.float32)
    grid_spec = pltpu.PrefetchScalarGridSpec(
        num_scalar_prefetch=2,
        grid=(B, NCH),
        in_specs=[
            pl.BlockSpec((1, H, DH), q_map),
            pl.BlockSpec((1, C_TC, KVH, DH), kv_map),
            pl.BlockSpec((1, C_TC, KVH, DH), kv_map),
            pl.BlockSpec((1, 1, 1, W), slots_map),
            pl.BlockSpec((1, 1, W, 1), slots_map),
            pl.BlockSpec((B, 1), sm_map),
            pl.BlockSpec((1, B), sm_map),
        ],
        out_specs=[
            pl.BlockSpec((1, H, DH), q_map),
            pl.BlockSpec((1, H, DH), q_map),
            pl.BlockSpec((1, H, DH), q_map),
            pl.BlockSpec((1, B, DH), q_map),
        ],
        scratch_shapes=[
            pltpu.VMEM((H, DH), jnp.float32),
            pltpu.VMEM((H, DH), jnp.float32),
            pltpu.VMEM((H, DH), jnp.float32),
            pltpu.VMEM((B, DH), jnp.float32),
        ],
    )
    return pl.pallas_call(
        _make_partial_body(half),
        grid_spec=grid_spec,
        out_shape=[sd, sd, sd, sdc],
        name=f"tc_flash_h{half}",
    )(lens, nactm1, q, gk, gv, slots4, slots4c, sm_c, sm_r)


def _merge_body(q_ref, kn_ref, vn_ref, m1_ref, l1_ref, a1_ref, c1_ref,
                m2_ref, l2_ref, a2_ref, c2_ref, o_ref):
    q_all = q_ref[0]
    knf = kn_ref[...].reshape(KVH * B, DH)
    vnf = vn_ref[...].reshape(KVH * B, DH)
    cand = lax.dot_general(q_all, knf, (((1,), (1,)), ((), ())),
                           preferred_element_type=jnp.float32) * SCALE
    hg = lax.broadcasted_iota(jnp.int32, (H, 1), 0) // GROUP
    rg = lax.broadcasted_iota(jnp.int32, (1, KVH * B), 1) // B
    cand = jnp.where(hg == rg, cand, NEG)

    m1 = m1_ref[0][:, 0:1]
    m2 = m2_ref[0][:, 0:1]
    m_fin = jnp.maximum(jnp.maximum(m1, m2),
                        jnp.max(cand, axis=1, keepdims=True))
    e_b = jnp.exp(cand - m_fin)

    cnt = c1_ref[0][:, 0:1] + c2_ref[0][:, 0:1]
    cnt_w = jnp.broadcast_to(cnt[None], (KVH, B, 1)).reshape(KVH * B, 1)
    l_b = lax.dot_general(e_b, cnt_w, (((1,), (0,)), ((), ())),
                          preferred_element_type=jnp.float32)
    acc_b = lax.dot_general(e_b, vnf * cnt_w, (((1,), (0,)), ((), ())),
                            preferred_element_type=jnp.float32)

    a1 = jnp.exp(m1 - m_fin)
    a2 = jnp.exp(m2 - m_fin)
    l_fin = l1_ref[0][:, 0:1] * a1 + l2_ref[0][:, 0:1] * a2 + l_b
    acc_fin = a1_ref[0] * a1 + a2_ref[0] * a2 + acc_b
    o_ref[0] = acc_fin / l_fin


def _merge(q, kn_t, vn_t, p1, p2):
    def bmap(b):
        return (b, 0, 0)

    def cmap(b):
        return (0, 0, 0)

    m1, l1, a1, c1 = p1
    m2, l2, a2, c2 = p2
    sb = pl.BlockSpec((1, H, DH), bmap)
    sc = pl.BlockSpec((1, B, DH), bmap)
    return pl.pallas_call(
        _merge_body,
        grid=(B,),
        in_specs=[sb, pl.BlockSpec((KVH, B, DH), cmap),
                  pl.BlockSpec((KVH, B, DH), cmap),
                  sb, sb, sb, sc, sb, sb, sb, sc],
        out_specs=sb,
        out_shape=jax.ShapeDtypeStruct((B, H, DH), jnp.float32),
        name="tc_merge",
    )(q, kn_t, vn_t, m1, l1, a1, c1, m2, l2, a2, c2)


def kernel(q, k, v, k_cache, v_cache, slot_mapping, active_slots, context_lens):
    lens = jnp.maximum(context_lens, 1).astype(jnp.int32)
    nactm1 = (lens - 1) // C_TC

    kn_t = jnp.transpose(k, (1, 0, 2))
    vn_t = jnp.transpose(v, (1, 0, 2))
    slots_exp = jnp.repeat(active_slots, KVH, axis=1)
    slots4 = slots_exp.reshape(B, NCHUNK, 1, W)
    slots4c = slots_exp.reshape(B, NCHUNK, W, 1)
    sm_i = slot_mapping.astype(jnp.int32)
    sm_c = sm_i.reshape(B, 1)
    sm_r = sm_i.reshape(1, B)

    gk1, gv1 = _sc_gather(k_cache, v_cache, active_slots, lens, 0)
    gk2, gv2 = _sc_gather(k_cache, v_cache, active_slots, lens, 1)
    p1 = _tc_partial(q, gk1, gv1, slots4[:, :NCH], slots4c[:, :NCH],
                     sm_c, sm_r, lens, nactm1, 0)
    p2 = _tc_partial(q, gk2, gv2, slots4[:, NCH:], slots4c[:, NCH:],
                     sm_c, sm_r, lens, nactm1, 1)
    return _merge(q, kn_t, vn_t, p1, p2)


# speedup vs baseline: 1.0812x; 1.0812x over previous
import functools

import jax
import jax.numpy as jnp
from jax import lax
from jax.experimental import pallas as pl
from jax.experimental.pallas import tpu as pltpu
from jax.experimental.pallas import tpu_sc as plsc

B = 16
S = 2048
H = 32
KVH = 8
DH = 128
SLOTS = 32768
SCALE = 0.08838834764831845
GROUP = H // KVH

C_SC = 32
ITEMS_PER_B = S // C_SC
NW = 32
ITEMS = B * ITEMS_PER_B
C_TC = 512
NCHUNK = S // C_TC

NEG = -1e30


def _sc_gather(k_cache, v_cache, active_slots, lens):
    mesh = plsc.VectorSubcoreMesh(
        core_axis_name="c", subcore_axis_name="s", num_cores=2, num_subcores=16
    )
    out_sd = jax.ShapeDtypeStruct((B, S, KVH, DH), jnp.float32)

    @functools.partial(
        pl.kernel,
        out_type=[out_sd, out_sd],
        mesh=mesh,
        scratch_types=[
            pltpu.VMEM((B,), jnp.int32),
            pltpu.VMEM((C_SC,), jnp.int32),
            pltpu.VMEM((C_SC, KVH, DH), jnp.float32),
            pltpu.VMEM((C_SC, KVH, DH), jnp.float32),
            pltpu.SemaphoreType.DMA,
            pltpu.SemaphoreType.DMA,
            pltpu.SemaphoreType.DMA,
            pltpu.SemaphoreType.DMA,
        ],
    )
    def sc_kernel(kc, vc, slots, lens_h, gk, gv,
                  lens_v, idx_v, rows0, rows1, sem_g0, sem_g1, sem_s0, sem_s1):
        wid = lax.axis_index("s") * 2 + lax.axis_index("c")
        pltpu.sync_copy(lens_h, lens_v)
        lens_vec = lens_v[...]

        items = [(b, r, cache)
                 for b in range(B)
                 for r in range(ITEMS_PER_B // NW)
                 for cache in (0, 1)]

        def active(i):
            b, r, _ = items[i]
            return (r * NW + wid) * C_SC < lens_vec[b]

        rows = (rows0, rows1)
        sem_g = (sem_g0, sem_g1)
        sem_s = (sem_s0, sem_s1)

        def drain(par, dst):
            pltpu.make_async_copy(
                rows[par], dst.at[0, pl.ds(0, C_SC)], sem_s[par]).wait()

        for i, (b, r, cache) in enumerate(items):
            if i >= 2:
                @pl.when(active(i - 2))
                def _(par=cache, dst=(gk, gv)[cache]):
                    drain(par, dst)

            @pl.when(active(i))
            def _(b=b, r=r, cache=cache):
                j = r * NW + wid
                if cache == 0:
                    pltpu.sync_copy(slots.at[b, pl.ds(j * C_SC, C_SC)], idx_v)
                src = (kc, vc)[cache]
                dst = (gk, gv)[cache]
                pltpu.async_copy(src.at[idx_v], rows[cache],
                                 sem_g[cache]).wait()
                pltpu.async_copy(rows[cache],
                                 dst.at[b, pl.ds(j * C_SC, C_SC)],
                                 sem_s[cache])

        for i in (len(items) - 2, len(items) - 1):
            @pl.when(active(i))
            def _(par=items[i][2], dst=(gk, gv)[items[i][2]]):
                drain(par, dst)

    return sc_kernel(k_cache, v_cache, active_slots, lens)


W = KVH * C_TC


def _tc_body(lens_ref, nactm1_ref, q_ref, k_ref, v_ref, kn_ref, vn_ref,
             slotsr_ref, slotsc_ref, smc_ref, smr_ref, o_ref,
             m_scr, l_scr, acc_scr, cnt_scr):
    b = pl.program_id(0)
    c = pl.program_id(1)

    @pl.when(c == 0)
    def _():
        m_scr[...] = jnp.full((H, DH), NEG, jnp.float32)
        l_scr[...] = jnp.zeros((H, DH), jnp.float32)
        acc_scr[...] = jnp.zeros((H, DH), jnp.float32)
        cnt_scr[...] = jnp.zeros((B, DH), jnp.float32)

    @pl.when(c <= nactm1_ref[b])
    def _():
        lb = lens_ref[b]
        q_all = q_ref[0]
        kflat = k_ref[0].reshape(W, DH)
        vflat = v_ref[0].reshape(W, DH)
        sm_c = smc_ref[...]
        sm_r = smr_ref[...]
        slots_row = slotsr_ref[0, 0]
        slots_col = slotsc_ref[0, 0]

        match16 = sm_c == slots_row
        validr = (c * C_TC
                  + lax.broadcasted_iota(jnp.int32, (1, W), 1) // KVH) < lb
        cnt_add = jnp.sum(jnp.where(match16 & validr, 1.0 / KVH, 0.0),
                          axis=1, keepdims=True)
        cnt_scr[...] = cnt_scr[...] + jnp.broadcast_to(cnt_add, (B, DH))

        newr = jnp.max(match16.astype(jnp.float32), axis=0, keepdims=True)
        keep_row = jnp.logical_and(validr, newr < 0.5)
        hg = lax.broadcasted_iota(jnp.int32, (H, 1), 0) // GROUP
        jg = lax.broadcasted_iota(jnp.int32, (1, W), 1) % KVH
        smask = jnp.logical_and(hg == jg, keep_row)

        matchc = jnp.max((slots_col == sm_r).astype(jnp.float32),
                         axis=1, keepdims=True)
        validc = (c * C_TC
                  + lax.broadcasted_iota(jnp.int32, (W, 1), 0) // KVH) < lb
        keep_c = jnp.logical_and(validc, matchc < 0.5)
        v_use = jnp.where(keep_c, vflat, 0.0)

        s = lax.dot_general(q_all, kflat, (((1,), (1,)), ((), ())),
                            preferred_element_type=jnp.float32) * SCALE
        s = jnp.where(smask, s, NEG)

        m_old = m_scr[:, 0:1]
        m_new = jnp.maximum(m_old, jnp.max(s, axis=1, keepdims=True))
        alpha = jnp.exp(m_old - m_new)
        p = jnp.where(smask, jnp.exp(s - m_new), 0.0)

        l_new = l_scr[:, 0:1] * alpha + jnp.sum(p, axis=1, keepdims=True)
        acc = acc_scr[...] * alpha + lax.dot_general(
            p, v_use, (((1,), (0,)), ((), ())),
            preferred_element_type=jnp.float32)

        m_scr[...] = jnp.broadcast_to(m_new, (H, DH))
        l_scr[...] = jnp.broadcast_to(l_new, (H, DH))
        acc_scr[...] = acc

    @pl.when(c == nactm1_ref[b])
    def _():
        q_all = q_ref[0]
        knf = kn_ref[...].reshape(KVH * B, DH)
        vnf = vn_ref[...].reshape(KVH * B, DH)
        cand = lax.dot_general(q_all, knf, (((1,), (1,)), ((), ())),
                               preferred_element_type=jnp.float32) * SCALE
        hg = lax.broadcasted_iota(jnp.int32, (H, 1), 0) // GROUP
        rg = lax.broadcasted_iota(jnp.int32, (1, KVH * B), 1) // B
        cand = jnp.where(hg == rg, cand, NEG)

        m_a = m_scr[:, 0:1]
        m_fin = jnp.maximum(m_a, jnp.max(cand, axis=1, keepdims=True))
        e_b = jnp.exp(cand - m_fin)

        cnt = cnt_scr[:, 0:1]
        cnt_w = jnp.broadcast_to(cnt[None], (KVH, B, 1)).reshape(KVH * B, 1)
        l_b = lax.dot_general(e_b, cnt_w, (((1,), (0,)), ((), ())),
                              preferred_element_type=jnp.float32)
        acc_b = lax.dot_general(e_b, vnf * cnt_w, (((1,), (0,)), ((), ())),
                                preferred_element_type=jnp.float32)

        alpha_a = jnp.exp(m_a - m_fin)
        l_fin = l_scr[:, 0:1] * alpha_a + l_b
        acc_fin = acc_scr[...] * alpha_a + acc_b
        o_ref[0] = acc_fin / l_fin


def _tc_attend(q, gk, gv, kn_t, vn_t, slots4, slots4c, sm_c, sm_r, lens, nactm1):
    def q_map(b, c, lens_ref, nactm1_ref):
        return (b, 0, 0)

    def kv_map(b, c, lens_ref, nactm1_ref):
        return (b, jnp.minimum(c, nactm1_ref[b]), 0, 0)

    def kn_map(b, c, lens_ref, nactm1_ref):
        return (0, 0, 0)

    def slots_map(b, c, lens_ref, nactm1_ref):
        return (b, jnp.minimum(c, nactm1_ref[b]), 0, 0)

    def sm_map(b, c, lens_ref, nactm1_ref):
        return (0, 0)

    grid_spec = pltpu.PrefetchScalarGridSpec(
        num_scalar_prefetch=2,
        grid=(B, NCHUNK),
        in_specs=[
            pl.BlockSpec((1, H, DH), q_map),
            pl.BlockSpec((1, C_TC, KVH, DH), kv_map),
            pl.BlockSpec((1, C_TC, KVH, DH), kv_map),
            pl.BlockSpec((KVH, B, DH), kn_map),
            pl.BlockSpec((KVH, B, DH), kn_map),
            pl.BlockSpec((1, 1, 1, W), slots_map),
            pl.BlockSpec((1, 1, W, 1), slots_map),
            pl.BlockSpec((B, 1), sm_map),
            pl.BlockSpec((1, B), sm_map),
        ],
        out_specs=pl.BlockSpec((1, H, DH), q_map),
        scratch_shapes=[
            pltpu.VMEM((H, DH), jnp.float32),
            pltpu.VMEM((H, DH), jnp.float32),
            pltpu.VMEM((H, DH), jnp.float32),
            pltpu.VMEM((B, DH), jnp.float32),
        ],
    )
    return pl.pallas_call(
        _tc_body,
        grid_spec=grid_spec,
        out_shape=jax.ShapeDtypeStruct((B, H, DH), jnp.float32),
    )(lens, nactm1, q, gk, gv, kn_t, vn_t, slots4, slots4c, sm_c, sm_r)


def kernel(q, k, v, k_cache, v_cache, slot_mapping, active_slots, context_lens):
    lens = jnp.maximum(context_lens, 1).astype(jnp.int32)
    nactm1 = (lens - 1) // C_TC

    gk, gv = _sc_gather(k_cache, v_cache, active_slots, lens)

    kn_t = jnp.transpose(k, (1, 0, 2))
    vn_t = jnp.transpose(v, (1, 0, 2))
    slots_exp = jnp.repeat(active_slots, KVH, axis=1)
    slots4 = slots_exp.reshape(B, NCHUNK, 1, W)
    slots4c = slots_exp.reshape(B, NCHUNK, W, 1)
    sm_i = slot_mapping.astype(jnp.int32)
    sm_c = sm_i.reshape(B, 1)
    sm_r = sm_i.reshape(1, B)

    return _tc_attend(q, gk, gv, kn_t, vn_t, slots4, slots4c, sm_c, sm_r,
                      lens, nactm1)
